# baseline (speedup 1.0000x reference)
import functools
import jax
import jax.numpy as jnp
from jax import lax
from jax.experimental import pallas as pl
from jax.experimental.pallas import tpu as pltpu
from jax.experimental.pallas import tpu_sc as plsc

_K = 100000
_B = 16384
_D = 64
_NW = 32
_CH = 512
_NMS = (_K - _B) // (_NW * _CH)
_REM_BASE = _B + _NMS * _NW * _CH
_REM = _K - _REM_BASE
_RT = _REM - 3 * _CH

_mesh = plsc.VectorSubcoreMesh(core_axis_name="c", subcore_axis_name="s")


@functools.partial(
    pl.kernel,
    mesh=_mesh,
    out_type=jax.ShapeDtypeStruct((_D, _K), jnp.float32),
    scratch_types=[
        pltpu.VMEM((_D, _CH), jnp.float32),
        pltpu.SemaphoreType.DMA,
    ],
)
def _sc_push(mem_hbm, val_hbm, out_hbm, buf, sem):
    w = lax.axis_index("s") * 2 + lax.axis_index("c")

    def cols(ref, base, n):
        return ref.at[:, pl.ds(pl.multiple_of(base, 128), n)]

    vbase = w * _CH
    pltpu.async_copy(cols(val_hbm, vbase, _CH), buf, sem).wait()
    pltpu.async_copy(buf, cols(out_hbm, vbase, _CH), sem).wait()

    for g in range(_NMS):
        base = _B + (g * _NW + w) * _CH
        pltpu.async_copy(cols(mem_hbm, base, _CH), buf, sem).wait()
        pltpu.async_copy(buf, cols(out_hbm, base, _CH), sem).wait()

    @pl.when(w < _REM // 128)
    def _():
        base = _REM_BASE + w * 128
        pltpu.async_copy(cols(mem_hbm, base, 128), buf.at[:, pl.ds(0, 128)], sem).wait()
        pltpu.async_copy(buf.at[:, pl.ds(0, 128)], cols(out_hbm, base, 128), sem).wait()


def kernel(mem, value):
    return _sc_push(mem.T, value.T).T


# speedup vs baseline: 1.0697x; 1.0697x over previous
import functools
import jax
import jax.numpy as jnp
from jax import lax
from jax.experimental import pallas as pl
from jax.experimental.pallas import tpu as pltpu
from jax.experimental.pallas import tpu_sc as plsc

_K = 100000
_B = 16384
_D = 64
_NW = 32
_CH = 512
_NMS = (_K - _B) // (_NW * _CH)
_REM_BASE = _B + _NMS * _NW * _CH
_REM = _K - _REM_BASE
_NSLOT = 3

_mesh = plsc.VectorSubcoreMesh(core_axis_name="c", subcore_axis_name="s")


@functools.partial(
    pl.kernel,
    mesh=_mesh,
    out_type=jax.ShapeDtypeStruct((_D, _K), jnp.float32),
    scratch_types=[
        pltpu.VMEM((_D, _CH), jnp.float32),
        pltpu.VMEM((_D, _CH), jnp.float32),
        pltpu.VMEM((_D, _CH), jnp.float32),
        pltpu.SemaphoreType.DMA,
        pltpu.SemaphoreType.DMA,
        pltpu.SemaphoreType.DMA,
        pltpu.SemaphoreType.DMA,
        pltpu.SemaphoreType.DMA,
        pltpu.SemaphoreType.DMA,
    ],
)
def _sc_push(mem_hbm, val_hbm, out_hbm, b0, b1, b2, r0, r1, r2, w0, w1, w2):
    bufs = [b0, b1, b2]
    rsem = [r0, r1, r2]
    wsem = [w0, w1, w2]
    w = lax.axis_index("s") * 2 + lax.axis_index("c")

    def cslice(ref, base):
        return ref.at[:, pl.ds(pl.multiple_of(base, 128), _CH)]

    chunks = [(val_hbm, w * _CH)]
    for g in range(_NMS):
        chunks.append((mem_hbm, _B + (g * _NW + w) * _CH))
    cn = len(chunks)

    rcp, wcp = {}, {}

    def start_write(j):
        rcp[j].wait()
        s = j % _NSLOT
        wcp[j] = pltpu.async_copy(bufs[s], cslice(out_hbm, chunks[j][1]), wsem[s])

    for i in range(cn):
        s = i % _NSLOT
        if i - _NSLOT >= 0:
            wcp[i - _NSLOT].wait()
        src, base = chunks[i]
        rcp[i] = pltpu.async_copy(cslice(src, base), bufs[s], rsem[s])
        if i - (_NSLOT - 1) >= 0:
            start_write(i - (_NSLOT - 1))
    for j in range(max(0, cn - (_NSLOT - 1)), cn):
        start_write(j)
    for j in range(max(0, cn - _NSLOT), cn):
        wcp[j].wait()

    @pl.when(w < _REM // 128)
    def _():
        base = pl.multiple_of(_REM_BASE + w * 128, 128)
        pltpu.async_copy(
            mem_hbm.at[:, pl.ds(base, 128)], b0.at[:, pl.ds(0, 128)], r0
        ).wait()
        pltpu.async_copy(
            b0.at[:, pl.ds(0, 128)], out_hbm.at[:, pl.ds(base, 128)], w0
        ).wait()


def kernel(mem, value):
    out = _sc_push(mem.T, value.T).T
    tail = jax.lax.slice(mem, (_K - 32, 0), (_K, _D))
    return jax.lax.dynamic_update_slice(out, tail, (_K - 32, 0))


# speedup vs baseline: 2.2815x; 2.1328x over previous
import jax
import jax.numpy as jnp
from jax.experimental import pallas as pl
from jax.experimental.pallas import tpu as pltpu

_K = 100000
_B = 16384
_D = 64
_CB = 16384
_VB = _B // _CB
_NB = pl.cdiv(_K, _CB)


def _push_body(mem_ref, val_ref, out_ref):
    i = pl.program_id(0)

    @pl.when(i < _VB)
    def _():
        out_ref[...] = val_ref[...]

    @pl.when(i >= _VB)
    def _():
        out_ref[...] = mem_ref[...]


def kernel(mem, value):
    out_t = pl.pallas_call(
        _push_body,
        grid=(_NB,),
        in_specs=[
            pl.BlockSpec((_D, _CB), lambda i: (0, jnp.maximum(i, _VB))),
            pl.BlockSpec((_D, _CB), lambda i: (0, jnp.minimum(i, _VB - 1))),
        ],
        out_specs=pl.BlockSpec((_D, _CB), lambda i: (0, i)),
        out_shape=jax.ShapeDtypeStruct((_D, _K), jnp.float32),
    )(mem.T, value.T)
    return out_t.T
